# baseline (speedup 1.0000x reference)
import functools

import jax
import jax.numpy as jnp
from jax import lax
from jax.experimental import pallas as pl
from jax.experimental.pallas import tpu as pltpu
from jax.experimental.pallas import tpu_sc as plsc

_LANES = 16
_MINOR = 128


def _make_mf_kernel(B, K, num_cores, num_subcores):
    NW = num_cores * num_subcores
    bpw = B // NW
    cpw = min(bpw, 256)
    n_chunks = bpw // cpw
    n_groups = cpw // _LANES

    mesh = plsc.VectorSubcoreMesh(core_axis_name="c", subcore_axis_name="s")

    @functools.partial(
        pl.kernel,
        out_type=jax.ShapeDtypeStruct((B,), jnp.float32),
        mesh=mesh,
        scratch_types=[
            pltpu.VMEM((bpw,), jnp.int32),
            pltpu.VMEM((bpw,), jnp.int32),
            pltpu.VMEM((bpw,), jnp.int32),
            pltpu.VMEM((bpw,), jnp.int32),
            pltpu.VMEM((cpw, _MINOR), jnp.float32),
            pltpu.VMEM((cpw, _MINOR), jnp.float32),
            pltpu.VMEM((bpw,), jnp.float32),
            pltpu.SemaphoreType.DMA,
        ],
        compiler_params=pltpu.CompilerParams(needs_layout_passes=False),
    )
    def mf_kernel(upad_hbm, vpad_hbm, uoff_hbm, voff_hbm, wb_hbm, hb_hbm,
                  out_hbm, upad_v, vpad_v, uoff_v, voff_v, urows_v, vrows_v,
                  out_v, sem):
        wid = lax.axis_index("s") * num_cores + lax.axis_index("c")
        base = wid * bpw

        pltpu.sync_copy(upad_hbm.at[pl.ds(base, bpw)], upad_v)
        pltpu.sync_copy(vpad_hbm.at[pl.ds(base, bpw)], vpad_v)
        pltpu.sync_copy(uoff_hbm.at[pl.ds(base, bpw)], uoff_v)
        pltpu.sync_copy(voff_hbm.at[pl.ds(base, bpw)], voff_v)

        lanes = lax.iota(jnp.int32, _LANES)

        for chunk in range(n_chunks):
            cbase = chunk * cpw
            cu = pltpu.async_copy(
                wb_hbm.at[upad_v.at[pl.ds(cbase, cpw)]], urows_v, sem)
            cv = pltpu.async_copy(
                hb_hbm.at[vpad_v.at[pl.ds(cbase, cpw)]], vrows_v, sem)
            cu.wait()
            cv.wait()

            def body(g, carry):
                slots = g * _LANES + lanes
                off_u = uoff_v[pl.ds(cbase + g * _LANES, _LANES)]
                off_v = voff_v[pl.ds(cbase + g * _LANES, _LANES)]
                acc = jnp.zeros((_LANES,), jnp.float32)
                for c in range(K):
                    uc = plsc.load_gather(urows_v, [slots, off_u + c])
                    vc = plsc.load_gather(vrows_v, [slots, off_v + c])
                    acc = acc + uc * vc
                sig = 1.0 / (1.0 + jnp.exp(-acc))
                out_v[pl.ds(cbase + g * _LANES, _LANES)] = sig
                return carry

            lax.fori_loop(0, n_groups, body, 0)

        pltpu.sync_copy(out_v, out_hbm.at[pl.ds(base, bpw)])

    return mf_kernel


def kernel(x, W, H):
    B = x.shape[0]
    K = W.shape[1]
    rows_per_block = _MINOR // K

    info = plsc.get_sparse_core_info()

    user_idx = x[:, 0]
    item_idx = x[:, 1]
    u_pad = user_idx // rows_per_block
    v_pad = item_idx // rows_per_block
    u_off = (user_idx % rows_per_block) * K
    v_off = (item_idx % rows_per_block) * K

    Wb = W.reshape(-1, _MINOR)
    Hb = H.reshape(-1, _MINOR)

    mf = _make_mf_kernel(B, K, info.num_cores, info.num_subcores)
    return mf(u_pad, v_pad, u_off, v_off, Wb, Hb)
